# baseline (speedup 1.0000x reference)
import jax
import jax.numpy as jnp
from jax.experimental import pallas as pl
from jax.experimental.pallas import tpu as pltpu

N = 10000
D_IN = 128
D_OUT = 128
BM = 200


def _gcn_kernel(seq_ref, w_ref, adj_ref, bias_ref, pw_ref, out_ref, fts_ref):
    @pl.when(pl.program_id(0) == 0)
    def _compute_fts():
        s = seq_ref[...].astype(jnp.bfloat16)
        w = w_ref[...].astype(jnp.bfloat16)
        fts = jax.lax.dot_general(
            s, w, (((1,), (1,)), ((), ())), preferred_element_type=jnp.float32
        )
        fts_ref[...] = fts.astype(jnp.bfloat16)

    a = adj_ref[...].astype(jnp.bfloat16)
    acc = jnp.dot(a, fts_ref[...], preferred_element_type=jnp.float32)
    acc = acc + bias_ref[...]
    out_ref[...] = jnp.where(acc >= 0, acc, pw_ref[...] * acc)


def kernel(seq, adj, W, bias, prelu_w):
    seq2 = seq[0]
    adj2 = adj[0]
    bias2 = bias.reshape(1, D_OUT)
    pw2 = jnp.broadcast_to(prelu_w.reshape(1, 1), (1, D_OUT))

    out = pl.pallas_call(
        _gcn_kernel,
        grid=(N // BM,),
        in_specs=[
            pl.BlockSpec((N, D_IN), lambda i: (0, 0)),
            pl.BlockSpec((D_OUT, D_IN), lambda i: (0, 0)),
            pl.BlockSpec((BM, N), lambda i: (i, 0)),
            pl.BlockSpec((1, D_OUT), lambda i: (0, 0)),
            pl.BlockSpec((1, D_OUT), lambda i: (0, 0)),
        ],
        out_specs=pl.BlockSpec((BM, D_OUT), lambda i: (i, 0)),
        out_shape=jax.ShapeDtypeStruct((N, D_OUT), jnp.float32),
        scratch_shapes=[pltpu.VMEM((N, D_OUT), jnp.bfloat16)],
        compiler_params=pltpu.CompilerParams(
            dimension_semantics=("arbitrary",),
        ),
    )(seq2, W, adj2, bias2, pw2)

    return out[None]


# speedup vs baseline: 1.0062x; 1.0062x over previous
import jax
import jax.numpy as jnp
from jax.experimental import pallas as pl
from jax.experimental.pallas import tpu as pltpu

N = 10000
D_IN = 128
D_OUT = 128
BM = 400


def _gcn_kernel(seq_ref, w_ref, adj_ref, bias_ref, pw_ref, out_ref, fts_ref):
    @pl.when(pl.program_id(0) == 0)
    def _compute_fts():
        s = seq_ref[...].astype(jnp.bfloat16)
        w = w_ref[...].astype(jnp.bfloat16)
        fts = jax.lax.dot_general(
            s, w, (((1,), (1,)), ((), ())), preferred_element_type=jnp.float32
        )
        fts_ref[...] = fts.astype(jnp.bfloat16)

    a = adj_ref[...].astype(jnp.bfloat16)
    acc = jnp.dot(a, fts_ref[...], preferred_element_type=jnp.float32)
    acc = acc + bias_ref[...]
    out_ref[...] = jnp.where(acc >= 0, acc, pw_ref[...] * acc)


def kernel(seq, adj, W, bias, prelu_w):
    seq2 = seq[0]
    adj2 = adj[0]
    bias2 = bias.reshape(1, D_OUT)
    pw2 = jnp.broadcast_to(prelu_w.reshape(1, 1), (1, D_OUT))

    out = pl.pallas_call(
        _gcn_kernel,
        grid=(N // BM,),
        in_specs=[
            pl.BlockSpec((N, D_IN), lambda i: (0, 0)),
            pl.BlockSpec((D_OUT, D_IN), lambda i: (0, 0)),
            pl.BlockSpec((BM, N), lambda i: (i, 0)),
            pl.BlockSpec((1, D_OUT), lambda i: (0, 0)),
            pl.BlockSpec((1, D_OUT), lambda i: (0, 0)),
        ],
        out_specs=pl.BlockSpec((BM, D_OUT), lambda i: (i, 0)),
        out_shape=jax.ShapeDtypeStruct((N, D_OUT), jnp.float32),
        scratch_shapes=[pltpu.VMEM((N, D_OUT), jnp.bfloat16)],
        compiler_params=pltpu.CompilerParams(
            dimension_semantics=("arbitrary",),
        ),
    )(seq2, W, adj2, bias2, pw2)

    return out[None]
